# Initial kernel scaffold; baseline (speedup 1.0000x reference)
import jax
import jax.numpy as jnp
from jax.experimental import pallas as pl


def kernel(node_features, edge_features, edge_index, We1, be1, We2, be2, Wn1, bn1, Wn2, bn2):
    raise NotImplementedError("write your pallas kernel here")


# speedup vs baseline: 3.5018x; 3.5018x over previous
import functools

import jax
import jax.numpy as jnp
from jax import lax
from jax.experimental import pallas as pl
from jax.experimental.pallas import tpu as pltpu
from jax.experimental.pallas import tpu_sc as plsc

N_NODES = 10000
N_EDGES = 320000
NODE_DIM = 128
EDGE_DIM = 64
HIDDEN = 128

NC = 2
NS = 16
NW = NC * NS

GWIN = 128
GRID = N_EDGES // GWIN


def _proj_body(n_ref, ws_ref, wd_ref, b1_ref, ps_ref, pd_ref):
    n = n_ref[...]
    ps_ref[...] = jnp.dot(n, ws_ref[...], preferred_element_type=jnp.float32) + b1_ref[...]
    pd_ref[...] = jnp.dot(n, wd_ref[...], preferred_element_type=jnp.float32)


def _proj(n, ws, wd, b1):
    R = 2000
    return pl.pallas_call(
        _proj_body,
        grid=(N_NODES // R,),
        in_specs=[
            pl.BlockSpec((R, NODE_DIM), lambda i: (i, 0)),
            pl.BlockSpec((NODE_DIM, HIDDEN), lambda i: (0, 0)),
            pl.BlockSpec((NODE_DIM, HIDDEN), lambda i: (0, 0)),
            pl.BlockSpec((1, HIDDEN), lambda i: (0, 0)),
        ],
        out_specs=[
            pl.BlockSpec((R, HIDDEN), lambda i: (i, 0)),
            pl.BlockSpec((R, HIDDEN), lambda i: (i, 0)),
        ],
        out_shape=[
            jax.ShapeDtypeStruct((N_NODES, HIDDEN), jnp.float32),
            jax.ShapeDtypeStruct((N_NODES, HIDDEN), jnp.float32),
        ],
    )(n, ws, wd, b1)


def _edge_body(e_ref, gs_ref, gd_ref, w1_ref, w2_ref, b2_ref, o_ref):
    e = e_ref[...][:, :EDGE_DIM]
    h = jnp.dot(e, w1_ref[...], preferred_element_type=jnp.float32)
    h = jnp.maximum(h + gs_ref[...] + gd_ref[...], 0.0)
    o = e + jnp.dot(h, w2_ref[...], preferred_element_type=jnp.float32) + b2_ref[...]
    o_ref[...] = jnp.concatenate([o, jnp.zeros_like(o)], axis=1)


def _edge_mlp(e, gs, gd, w1e, w2, b2):
    R = 3200
    ew = e.shape[1]
    return pl.pallas_call(
        _edge_body,
        grid=(N_EDGES // R,),
        in_specs=[
            pl.BlockSpec((R, ew), lambda i: (i, 0)),
            pl.BlockSpec((R, HIDDEN), lambda i: (i, 0)),
            pl.BlockSpec((R, HIDDEN), lambda i: (i, 0)),
            pl.BlockSpec((EDGE_DIM, HIDDEN), lambda i: (0, 0)),
            pl.BlockSpec((HIDDEN, EDGE_DIM), lambda i: (0, 0)),
            pl.BlockSpec((1, EDGE_DIM), lambda i: (0, 0)),
        ],
        out_specs=pl.BlockSpec((R, 2 * EDGE_DIM), lambda i: (i, 0)),
        out_shape=jax.ShapeDtypeStruct((N_EDGES, 2 * EDGE_DIM), jnp.float32),
    )(e, gs, gd, w1e, w2, b2)


def _node_body(n_ref, a0_ref, a1_ref, w1n_ref, w1a_ref, b1_ref, w2_ref, b2_ref, o_ref):
    n = n_ref[...]
    agg = a0_ref[...] + a1_ref[...]
    h = (jnp.dot(n, w1n_ref[...], preferred_element_type=jnp.float32)
         + jnp.dot(agg, w1a_ref[...], preferred_element_type=jnp.float32)
         + b1_ref[...])
    h = jnp.maximum(h, 0.0)
    o_ref[...] = n + jnp.dot(h, w2_ref[...], preferred_element_type=jnp.float32) + b2_ref[...]


def _node_mlp(n, a0, a1, w1n, w1a, b1, w2, b2):
    R = 2000
    return pl.pallas_call(
        _node_body,
        grid=(N_NODES // R,),
        in_specs=[
            pl.BlockSpec((R, NODE_DIM), lambda i: (i, 0)),
            pl.BlockSpec((R, 2 * EDGE_DIM), lambda i: (i, 0)),
            pl.BlockSpec((R, 2 * EDGE_DIM), lambda i: (i, 0)),
            pl.BlockSpec((NODE_DIM, HIDDEN), lambda i: (0, 0)),
            pl.BlockSpec((2 * EDGE_DIM, HIDDEN), lambda i: (0, 0)),
            pl.BlockSpec((1, HIDDEN), lambda i: (0, 0)),
            pl.BlockSpec((HIDDEN, NODE_DIM), lambda i: (0, 0)),
            pl.BlockSpec((1, NODE_DIM), lambda i: (0, 0)),
        ],
        out_specs=pl.BlockSpec((R, NODE_DIM), lambda i: (i, 0)),
        out_shape=jax.ShapeDtypeStruct((N_NODES, NODE_DIM), jnp.float32),
    )(n, a0, a1, w1n, w1a, b1, w2, b2)


@functools.cache
def _sc_kernels():
    mesh = plsc.VectorSubcoreMesh(core_axis_name="c", subcore_axis_name="s")

    @functools.partial(
        pl.kernel,
        out_type=(
            jax.ShapeDtypeStruct((N_EDGES, HIDDEN), jnp.float32),
            jax.ShapeDtypeStruct((N_EDGES, HIDDEN), jnp.float32),
        ),
        mesh=mesh,
    )
    def sc_gather2(ps_hbm, pd_hbm, src_hbm, dst_hbm, gs_hbm, gd_hbm):
        def body(si_v, di_v, gs_v, gd_v):
            pltpu.sync_copy(ps_hbm.at[si_v.at[0]], gs_v)
            pltpu.sync_copy(pd_hbm.at[di_v.at[0]], gd_v)

        pltpu.emit_pipeline(
            body,
            grid=(GRID,),
            in_specs=[
                pl.BlockSpec((1, GWIN), lambda i: (0, i)),
                pl.BlockSpec((1, GWIN), lambda i: (0, i)),
            ],
            out_specs=[
                pl.BlockSpec((GWIN, HIDDEN), lambda i: (i, 0)),
                pl.BlockSpec((GWIN, HIDDEN), lambda i: (i, 0)),
            ],
            core_axis_name=("c", "s"),
            dimension_semantics=(pltpu.PARALLEL,),
        )(src_hbm, dst_hbm, gs_hbm, gd_hbm)

    @functools.partial(
        pl.kernel,
        out_type=jax.ShapeDtypeStruct((NC, N_NODES, 2 * EDGE_DIM), jnp.float32),
        mesh=mesh,
        scratch_types=[
            pltpu.VMEM_SHARED((N_NODES, 2 * EDGE_DIM), jnp.float32),
            pltpu.VMEM((GWIN,), jnp.int32),
            pltpu.VMEM((GWIN, 2 * EDGE_DIM), jnp.float32),
        ],
    )
    def sc_scatter_add(e_hbm, dst_hbm, zero_hbm, out_hbm, acc_sh, idx_v, rows_v):
        c = lax.axis_index("c")
        s = lax.axis_index("s")
        wid = s * NC + c
        rows = 624
        tail_off = NS * rows
        tail = N_NODES - tail_off

        pltpu.sync_copy(zero_hbm.at[pl.ds(s * rows, rows)],
                        acc_sh.at[pl.ds(s * rows, rows)])

        @pl.when(s == 0)
        def _():
            pltpu.sync_copy(zero_hbm.at[pl.ds(tail_off, tail)],
                            acc_sh.at[pl.ds(tail_off, tail)])

        plsc.subcore_barrier()

        @pl.loop(0, GRID // NW + 1)
        def _(i):
            k = i * NW + wid

            @pl.when(k < GRID)
            def _():
                pltpu.sync_copy(dst_hbm.at[pl.ds(k * GWIN, GWIN)], idx_v)
                pltpu.sync_copy(e_hbm.at[pl.ds(k * GWIN, GWIN)], rows_v)
                pltpu.sync_copy(rows_v, acc_sh.at[idx_v], add=True)

        plsc.subcore_barrier()
        pltpu.sync_copy(acc_sh.at[pl.ds(s * rows, rows)],
                        out_hbm.at[c].at[pl.ds(s * rows, rows)])

        @pl.when(s == 0)
        def _():
            pltpu.sync_copy(acc_sh.at[pl.ds(tail_off, tail)],
                            out_hbm.at[c].at[pl.ds(tail_off, tail)])

    return sc_gather2, sc_scatter_add


def kernel(node_features, edge_features, edge_index, We1, be1, We2, be2,
           Wn1, bn1, Wn2, bn2):
    n = node_features
    e = edge_features
    src = edge_index[0].astype(jnp.int32)
    dst = edge_index[1].astype(jnp.int32)
    src2 = src.reshape(1, N_EDGES)
    dst2 = dst.reshape(1, N_EDGES)
    zero = jnp.zeros((N_NODES, 2 * EDGE_DIM), jnp.float32)
    L = We1.shape[0]
    for l in range(L):
        w1e = We1[l, :EDGE_DIM]
        w1s = We1[l, EDGE_DIM:EDGE_DIM + NODE_DIM]
        w1d = We1[l, EDGE_DIM + NODE_DIM:]
        b1 = be1[l].reshape(1, HIDDEN)
        w2 = We2[l]
        b2 = be2[l].reshape(1, EDGE_DIM)
        w1a = jnp.pad(Wn1[l, NODE_DIM:], ((0, EDGE_DIM), (0, 0)))
        sc_gather2, sc_scatter_add = _sc_kernels()
        ps, pd = _proj(n, w1s, w1d, b1)
        gs, gd = sc_gather2(ps, pd, src2, dst2)
        e = _edge_mlp(e, gs, gd, w1e, w2, b2)
        agg = sc_scatter_add(e, dst, zero)
        n = _node_mlp(n, agg[0], agg[1],
                      Wn1[l, :NODE_DIM], w1a,
                      bn1[l].reshape(1, HIDDEN), Wn2[l],
                      bn2[l].reshape(1, NODE_DIM))
    return n


# speedup vs baseline: 4.0428x; 1.1545x over previous
import functools

import jax
import jax.numpy as jnp
from jax import lax
from jax.experimental import pallas as pl
from jax.experimental.pallas import tpu as pltpu
from jax.experimental.pallas import tpu_sc as plsc

N_NODES = 10000
N_EDGES = 320000
NODE_DIM = 128
EDGE_DIM = 64
HIDDEN = 128

NC = 2
NS = 16
NW = NC * NS

GWIN = 128
GRID = N_EDGES // GWIN


def _proj_body(n_ref, ws_ref, wd_ref, b1_ref, ps_ref, pd_ref):
    n = n_ref[...]
    ps_ref[...] = jnp.dot(n, ws_ref[...], preferred_element_type=jnp.float32) + b1_ref[...]
    pd_ref[...] = jnp.dot(n, wd_ref[...], preferred_element_type=jnp.float32)


def _proj(n, ws, wd, b1):
    R = 2000
    return pl.pallas_call(
        _proj_body,
        grid=(N_NODES // R,),
        in_specs=[
            pl.BlockSpec((R, NODE_DIM), lambda i: (i, 0)),
            pl.BlockSpec((NODE_DIM, HIDDEN), lambda i: (0, 0)),
            pl.BlockSpec((NODE_DIM, HIDDEN), lambda i: (0, 0)),
            pl.BlockSpec((1, HIDDEN), lambda i: (0, 0)),
        ],
        out_specs=[
            pl.BlockSpec((R, HIDDEN), lambda i: (i, 0)),
            pl.BlockSpec((R, HIDDEN), lambda i: (i, 0)),
        ],
        out_shape=[
            jax.ShapeDtypeStruct((N_NODES, HIDDEN), jnp.float32),
            jax.ShapeDtypeStruct((N_NODES, HIDDEN), jnp.float32),
        ],
    )(n, ws, wd, b1)


def _edge_body(e_ref, gs_ref, gd_ref, w1_ref, w2_ref, b2_ref, o_ref):
    e = e_ref[...][:, :EDGE_DIM]
    h = jnp.dot(e, w1_ref[...], preferred_element_type=jnp.float32)
    h = jnp.maximum(h + gs_ref[...] + gd_ref[...], 0.0)
    o = e + jnp.dot(h, w2_ref[...], preferred_element_type=jnp.float32) + b2_ref[...]
    o_ref[...] = jnp.concatenate([o, jnp.zeros_like(o)], axis=1)


def _edge_mlp(e, gs, gd, w1e, w2, b2):
    R = 3200
    ew = e.shape[1]
    return pl.pallas_call(
        _edge_body,
        grid=(N_EDGES // R,),
        in_specs=[
            pl.BlockSpec((R, ew), lambda i: (i, 0)),
            pl.BlockSpec((R, HIDDEN), lambda i: (i, 0)),
            pl.BlockSpec((R, HIDDEN), lambda i: (i, 0)),
            pl.BlockSpec((EDGE_DIM, HIDDEN), lambda i: (0, 0)),
            pl.BlockSpec((HIDDEN, EDGE_DIM), lambda i: (0, 0)),
            pl.BlockSpec((1, EDGE_DIM), lambda i: (0, 0)),
        ],
        out_specs=pl.BlockSpec((R, 2 * EDGE_DIM), lambda i: (i, 0)),
        out_shape=jax.ShapeDtypeStruct((N_EDGES, 2 * EDGE_DIM), jnp.float32),
    )(e, gs, gd, w1e, w2, b2)


def _node_body(n_ref, a0_ref, a1_ref, w1n_ref, w1a_ref, b1_ref, w2_ref, b2_ref, o_ref):
    n = n_ref[...]
    agg = a0_ref[...] + a1_ref[...]
    h = (jnp.dot(n, w1n_ref[...], preferred_element_type=jnp.float32)
         + jnp.dot(agg, w1a_ref[...], preferred_element_type=jnp.float32)
         + b1_ref[...])
    h = jnp.maximum(h, 0.0)
    o_ref[...] = n + jnp.dot(h, w2_ref[...], preferred_element_type=jnp.float32) + b2_ref[...]


def _node_mlp(n, a0, a1, w1n, w1a, b1, w2, b2):
    R = 2000
    return pl.pallas_call(
        _node_body,
        grid=(N_NODES // R,),
        in_specs=[
            pl.BlockSpec((R, NODE_DIM), lambda i: (i, 0)),
            pl.BlockSpec((R, 2 * EDGE_DIM), lambda i: (i, 0)),
            pl.BlockSpec((R, 2 * EDGE_DIM), lambda i: (i, 0)),
            pl.BlockSpec((NODE_DIM, HIDDEN), lambda i: (0, 0)),
            pl.BlockSpec((2 * EDGE_DIM, HIDDEN), lambda i: (0, 0)),
            pl.BlockSpec((1, HIDDEN), lambda i: (0, 0)),
            pl.BlockSpec((HIDDEN, NODE_DIM), lambda i: (0, 0)),
            pl.BlockSpec((1, NODE_DIM), lambda i: (0, 0)),
        ],
        out_specs=pl.BlockSpec((R, NODE_DIM), lambda i: (i, 0)),
        out_shape=jax.ShapeDtypeStruct((N_NODES, NODE_DIM), jnp.float32),
    )(n, a0, a1, w1n, w1a, b1, w2, b2)


@functools.cache
def _sc_kernels():
    mesh = plsc.VectorSubcoreMesh(core_axis_name="c", subcore_axis_name="s")

    @functools.partial(
        pl.kernel,
        out_type=(
            jax.ShapeDtypeStruct((N_EDGES, HIDDEN), jnp.float32),
            jax.ShapeDtypeStruct((N_EDGES, HIDDEN), jnp.float32),
        ),
        mesh=mesh,
    )
    def sc_gather2(ps_hbm, pd_hbm, src_hbm, dst_hbm, gs_hbm, gd_hbm):
        def body(si_v, di_v, gs_v, gd_v):
            pltpu.sync_copy(ps_hbm.at[si_v.at[0]], gs_v)
            pltpu.sync_copy(pd_hbm.at[di_v.at[0]], gd_v)

        pltpu.emit_pipeline(
            body,
            grid=(GRID,),
            in_specs=[
                pl.BlockSpec((1, GWIN), lambda i: (0, i)),
                pl.BlockSpec((1, GWIN), lambda i: (0, i)),
            ],
            out_specs=[
                pl.BlockSpec((GWIN, HIDDEN), lambda i: (i, 0)),
                pl.BlockSpec((GWIN, HIDDEN), lambda i: (i, 0)),
            ],
            core_axis_name=("c", "s"),
            dimension_semantics=(pltpu.PARALLEL,),
        )(src_hbm, dst_hbm, gs_hbm, gd_hbm)

    @functools.partial(
        pl.kernel,
        out_type=jax.ShapeDtypeStruct((NC, N_NODES, 2 * EDGE_DIM), jnp.float32),
        mesh=mesh,
        scratch_types=[
            pltpu.VMEM_SHARED((N_NODES, 2 * EDGE_DIM), jnp.float32),
            pltpu.VMEM((GWIN,), jnp.int32),
            pltpu.VMEM((GWIN, 2 * EDGE_DIM), jnp.float32),
            pltpu.VMEM((GWIN,), jnp.int32),
            pltpu.VMEM((GWIN, 2 * EDGE_DIM), jnp.float32),
            pltpu.SemaphoreType.DMA,
            pltpu.SemaphoreType.DMA,
            pltpu.SemaphoreType.DMA,
            pltpu.SemaphoreType.DMA,
        ],
    )
    def sc_scatter_add(e_hbm, dst_hbm, zero_hbm, out_hbm, acc_sh,
                       idx0_v, rows0_v, idx1_v, rows1_v,
                       si0, sr0, si1, sr1):
        c = lax.axis_index("c")
        s = lax.axis_index("s")
        wid = s * NC + c
        rows = 624
        tail_off = NS * rows
        tail = N_NODES - tail_off

        pltpu.sync_copy(zero_hbm.at[pl.ds(s * rows, rows)],
                        acc_sh.at[pl.ds(s * rows, rows)])

        @pl.when(s == 0)
        def _():
            pltpu.sync_copy(zero_hbm.at[pl.ds(tail_off, tail)],
                            acc_sh.at[pl.ds(tail_off, tail)])

        plsc.subcore_barrier()

        bufs = ((idx0_v, rows0_v, si0, sr0), (idx1_v, rows1_v, si1, sr1))

        def issue(j, idx_b, rows_b, si_b, sr_b):
            k = j * NW + wid
            pltpu.async_copy(dst_hbm.at[pl.ds(k * GWIN, GWIN)], idx_b, si_b)
            pltpu.async_copy(e_hbm.at[pl.ds(k * GWIN, GWIN)], rows_b, sr_b)

        issue(0, *bufs[0])
        issue(1, *bufs[1])

        @pl.loop(0, GRID // (2 * NW) + 1)
        def _(i):
            for b in range(2):
                j = 2 * i + b
                k = j * NW + wid
                idx_b, rows_b, si_b, sr_b = bufs[b]

                @pl.when(k < GRID)
                def _():
                    pltpu.make_async_copy(
                        dst_hbm.at[pl.ds(k * GWIN, GWIN)], idx_b, si_b).wait()
                    pltpu.make_async_copy(
                        e_hbm.at[pl.ds(k * GWIN, GWIN)], rows_b, sr_b).wait()
                    pltpu.sync_copy(rows_b, acc_sh.at[idx_b], add=True)

                    @pl.when((j + 2) * NW + wid < GRID)
                    def _():
                        issue(j + 2, idx_b, rows_b, si_b, sr_b)

        plsc.subcore_barrier()
        pltpu.sync_copy(acc_sh.at[pl.ds(s * rows, rows)],
                        out_hbm.at[c].at[pl.ds(s * rows, rows)])

        @pl.when(s == 0)
        def _():
            pltpu.sync_copy(acc_sh.at[pl.ds(tail_off, tail)],
                            out_hbm.at[c].at[pl.ds(tail_off, tail)])

    return sc_gather2, sc_scatter_add


def kernel(node_features, edge_features, edge_index, We1, be1, We2, be2,
           Wn1, bn1, Wn2, bn2):
    n = node_features
    e = edge_features
    src = edge_index[0].astype(jnp.int32)
    dst = edge_index[1].astype(jnp.int32)
    src2 = src.reshape(1, N_EDGES)
    dst2 = dst.reshape(1, N_EDGES)
    zero = jnp.zeros((N_NODES, 2 * EDGE_DIM), jnp.float32)
    L = We1.shape[0]
    for l in range(L):
        w1e = We1[l, :EDGE_DIM]
        w1s = We1[l, EDGE_DIM:EDGE_DIM + NODE_DIM]
        w1d = We1[l, EDGE_DIM + NODE_DIM:]
        b1 = be1[l].reshape(1, HIDDEN)
        w2 = We2[l]
        b2 = be2[l].reshape(1, EDGE_DIM)
        w1a = jnp.pad(Wn1[l, NODE_DIM:], ((0, EDGE_DIM), (0, 0)))
        sc_gather2, sc_scatter_add = _sc_kernels()
        ps, pd = _proj(n, w1s, w1d, b1)
        gs, gd = sc_gather2(ps, pd, src2, dst2)
        e = _edge_mlp(e, gs, gd, w1e, w2, b2)
        agg = sc_scatter_add(e, dst, zero)
        n = _node_mlp(n, agg[0], agg[1],
                      Wn1[l, :NODE_DIM], w1a,
                      bn1[l].reshape(1, HIDDEN), Wn2[l],
                      bn2[l].reshape(1, NODE_DIM))
    return n


# speedup vs baseline: 4.2697x; 1.0561x over previous
import functools

import jax
import jax.numpy as jnp
from jax import lax
from jax.experimental import pallas as pl
from jax.experimental.pallas import tpu as pltpu
from jax.experimental.pallas import tpu_sc as plsc

N_NODES = 10000
N_EDGES = 320000
NODE_DIM = 128
EDGE_DIM = 64
HIDDEN = 128

NC = 2
NS = 16
NW = NC * NS

GWIN = 128
GRID = N_EDGES // GWIN


def _proj_body(n_ref, ws_ref, wd_ref, b1_ref, ps_ref, pd_ref):
    n = n_ref[...]
    ps_ref[...] = jnp.dot(n, ws_ref[...], preferred_element_type=jnp.float32) + b1_ref[...]
    pd_ref[...] = jnp.dot(n, wd_ref[...], preferred_element_type=jnp.float32)


def _proj(n, ws, wd, b1):
    R = 2000
    return pl.pallas_call(
        _proj_body,
        grid=(N_NODES // R,),
        in_specs=[
            pl.BlockSpec((R, NODE_DIM), lambda i: (i, 0)),
            pl.BlockSpec((NODE_DIM, HIDDEN), lambda i: (0, 0)),
            pl.BlockSpec((NODE_DIM, HIDDEN), lambda i: (0, 0)),
            pl.BlockSpec((1, HIDDEN), lambda i: (0, 0)),
        ],
        out_specs=[
            pl.BlockSpec((R, HIDDEN), lambda i: (i, 0)),
            pl.BlockSpec((R, HIDDEN), lambda i: (i, 0)),
        ],
        out_shape=[
            jax.ShapeDtypeStruct((N_NODES, HIDDEN), jnp.float32),
            jax.ShapeDtypeStruct((N_NODES, HIDDEN), jnp.float32),
        ],
    )(n, ws, wd, b1)


def _edge_body(e_ref, gs_ref, gd_ref, w1_ref, w2_ref, b2_ref, o_ref):
    e = e_ref[...][:, :EDGE_DIM]
    h = jnp.dot(e, w1_ref[...], preferred_element_type=jnp.float32)
    h = jnp.maximum(h + gs_ref[...] + gd_ref[...], 0.0)
    o = e + jnp.dot(h, w2_ref[...], preferred_element_type=jnp.float32) + b2_ref[...]
    o_ref[...] = jnp.concatenate([o, jnp.zeros_like(o)], axis=1)


def _edge_mlp(e, gs, gd, w1e, w2, b2):
    R = 3200
    ew = e.shape[1]
    return pl.pallas_call(
        _edge_body,
        grid=(N_EDGES // R,),
        in_specs=[
            pl.BlockSpec((R, ew), lambda i: (i, 0)),
            pl.BlockSpec((R, HIDDEN), lambda i: (i, 0)),
            pl.BlockSpec((R, HIDDEN), lambda i: (i, 0)),
            pl.BlockSpec((EDGE_DIM, HIDDEN), lambda i: (0, 0)),
            pl.BlockSpec((HIDDEN, EDGE_DIM), lambda i: (0, 0)),
            pl.BlockSpec((1, EDGE_DIM), lambda i: (0, 0)),
        ],
        out_specs=pl.BlockSpec((R, 2 * EDGE_DIM), lambda i: (i, 0)),
        out_shape=jax.ShapeDtypeStruct((N_EDGES, 2 * EDGE_DIM), jnp.float32),
    )(e, gs, gd, w1e, w2, b2)


def _node_body(n_ref, a0_ref, a1_ref, w1n_ref, w1a_ref, b1_ref, w2_ref, b2_ref, o_ref):
    n = n_ref[...]
    agg = a0_ref[...] + a1_ref[...]
    h = (jnp.dot(n, w1n_ref[...], preferred_element_type=jnp.float32)
         + jnp.dot(agg, w1a_ref[...], preferred_element_type=jnp.float32)
         + b1_ref[...])
    h = jnp.maximum(h, 0.0)
    o_ref[...] = n + jnp.dot(h, w2_ref[...], preferred_element_type=jnp.float32) + b2_ref[...]


def _node_mlp(n, a0, a1, w1n, w1a, b1, w2, b2):
    R = 2000
    return pl.pallas_call(
        _node_body,
        grid=(N_NODES // R,),
        in_specs=[
            pl.BlockSpec((R, NODE_DIM), lambda i: (i, 0)),
            pl.BlockSpec((R, 2 * EDGE_DIM), lambda i: (i, 0)),
            pl.BlockSpec((R, 2 * EDGE_DIM), lambda i: (i, 0)),
            pl.BlockSpec((NODE_DIM, HIDDEN), lambda i: (0, 0)),
            pl.BlockSpec((2 * EDGE_DIM, HIDDEN), lambda i: (0, 0)),
            pl.BlockSpec((1, HIDDEN), lambda i: (0, 0)),
            pl.BlockSpec((HIDDEN, NODE_DIM), lambda i: (0, 0)),
            pl.BlockSpec((1, NODE_DIM), lambda i: (0, 0)),
        ],
        out_specs=pl.BlockSpec((R, NODE_DIM), lambda i: (i, 0)),
        out_shape=jax.ShapeDtypeStruct((N_NODES, NODE_DIM), jnp.float32),
    )(n, a0, a1, w1n, w1a, b1, w2, b2)


@functools.cache
def _sc_kernels():
    mesh = plsc.VectorSubcoreMesh(core_axis_name="c", subcore_axis_name="s")

    gather_scratch = []
    for _ in range(2):
        gather_scratch += [
            pltpu.VMEM((GWIN,), jnp.int32),
            pltpu.VMEM((GWIN,), jnp.int32),
            pltpu.VMEM((GWIN, HIDDEN), jnp.float32),
            pltpu.VMEM((GWIN, HIDDEN), jnp.float32),
        ] + [pltpu.SemaphoreType.DMA] * 6

    @functools.partial(
        pl.kernel,
        out_type=(
            jax.ShapeDtypeStruct((N_EDGES, HIDDEN), jnp.float32),
            jax.ShapeDtypeStruct((N_EDGES, HIDDEN), jnp.float32),
        ),
        mesh=mesh,
        scratch_types=gather_scratch,
    )
    def sc_gather2(ps_hbm, pd_hbm, src_hbm, dst_hbm, gs_hbm, gd_hbm, *scr):
        c = lax.axis_index("c")
        s = lax.axis_index("s")
        wid = s * NC + c
        bufs = (scr[:10], scr[10:])

        def issue_idx(j, buf):
            si_v, di_v = buf[0], buf[1]
            ssi, sdi = buf[4], buf[5]
            k = j * NW + wid
            pltpu.async_copy(src_hbm.at[pl.ds(k * GWIN, GWIN)], si_v, ssi)
            pltpu.async_copy(dst_hbm.at[pl.ds(k * GWIN, GWIN)], di_v, sdi)

        issue_idx(0, bufs[0])
        issue_idx(1, bufs[1])

        @pl.loop(0, GRID // (2 * NW) + 1)
        def _(i):
            for b in range(2):
                j = 2 * i + b
                k = j * GWIN * NW + wid * GWIN
                si_v, di_v, gs_v, gd_v, ssi, sdi, sgs, sgd, sws, swd = bufs[b]

                @pl.when(j * NW + wid < GRID)
                def _():
                    pltpu.make_async_copy(
                        src_hbm.at[pl.ds(k, GWIN)], si_v, ssi).wait()
                    pltpu.make_async_copy(
                        dst_hbm.at[pl.ds(k, GWIN)], di_v, sdi).wait()

                    @pl.when(i >= 1)
                    def _():
                        kprev = k - 2 * GWIN * NW
                        pltpu.make_async_copy(
                            gs_v, gs_hbm.at[pl.ds(kprev, GWIN)], sws).wait()
                        pltpu.make_async_copy(
                            gd_v, gd_hbm.at[pl.ds(kprev, GWIN)], swd).wait()

                    pltpu.async_copy(ps_hbm.at[si_v], gs_v, sgs)
                    pltpu.async_copy(pd_hbm.at[di_v], gd_v, sgd)
                    pltpu.make_async_copy(ps_hbm.at[si_v], gs_v, sgs).wait()
                    pltpu.make_async_copy(pd_hbm.at[di_v], gd_v, sgd).wait()
                    pltpu.async_copy(gs_v, gs_hbm.at[pl.ds(k, GWIN)], sws)
                    pltpu.async_copy(gd_v, gd_hbm.at[pl.ds(k, GWIN)], swd)

                    @pl.when((j + 2) * NW + wid < GRID)
                    def _():
                        issue_idx(j + 2, bufs[b])

        for b in range(2):
            si_v, di_v, gs_v, gd_v, ssi, sdi, sgs, sgd, sws, swd = bufs[b]
            pltpu.make_async_copy(gs_v, gs_hbm.at[pl.ds(0, GWIN)], sws).wait()
            pltpu.make_async_copy(gd_v, gd_hbm.at[pl.ds(0, GWIN)], swd).wait()

    @functools.partial(
        pl.kernel,
        out_type=jax.ShapeDtypeStruct((NC, N_NODES, 2 * EDGE_DIM), jnp.float32),
        mesh=mesh,
        scratch_types=[
            pltpu.VMEM_SHARED((N_NODES, 2 * EDGE_DIM), jnp.float32),
            pltpu.VMEM((GWIN,), jnp.int32),
            pltpu.VMEM((GWIN, 2 * EDGE_DIM), jnp.float32),
            pltpu.VMEM((GWIN,), jnp.int32),
            pltpu.VMEM((GWIN, 2 * EDGE_DIM), jnp.float32),
            pltpu.SemaphoreType.DMA,
            pltpu.SemaphoreType.DMA,
            pltpu.SemaphoreType.DMA,
            pltpu.SemaphoreType.DMA,
        ],
    )
    def sc_scatter_add(e_hbm, dst_hbm, zero_hbm, out_hbm, acc_sh,
                       idx0_v, rows0_v, idx1_v, rows1_v,
                       si0, sr0, si1, sr1):
        c = lax.axis_index("c")
        s = lax.axis_index("s")
        wid = s * NC + c
        rows = 624
        tail_off = NS * rows
        tail = N_NODES - tail_off

        pltpu.sync_copy(zero_hbm.at[pl.ds(s * rows, rows)],
                        acc_sh.at[pl.ds(s * rows, rows)])

        @pl.when(s == 0)
        def _():
            pltpu.sync_copy(zero_hbm.at[pl.ds(tail_off, tail)],
                            acc_sh.at[pl.ds(tail_off, tail)])

        plsc.subcore_barrier()

        bufs = ((idx0_v, rows0_v, si0, sr0), (idx1_v, rows1_v, si1, sr1))

        def issue(j, idx_b, rows_b, si_b, sr_b):
            k = j * NW + wid
            pltpu.async_copy(dst_hbm.at[pl.ds(k * GWIN, GWIN)], idx_b, si_b)
            pltpu.async_copy(e_hbm.at[pl.ds(k * GWIN, GWIN)], rows_b, sr_b)

        issue(0, *bufs[0])
        issue(1, *bufs[1])

        @pl.loop(0, GRID // (2 * NW) + 1)
        def _(i):
            for b in range(2):
                j = 2 * i + b
                k = j * NW + wid
                idx_b, rows_b, si_b, sr_b = bufs[b]

                @pl.when(k < GRID)
                def _():
                    pltpu.make_async_copy(
                        dst_hbm.at[pl.ds(k * GWIN, GWIN)], idx_b, si_b).wait()
                    pltpu.make_async_copy(
                        e_hbm.at[pl.ds(k * GWIN, GWIN)], rows_b, sr_b).wait()
                    pltpu.sync_copy(rows_b, acc_sh.at[idx_b], add=True)

                    @pl.when((j + 2) * NW + wid < GRID)
                    def _():
                        issue(j + 2, idx_b, rows_b, si_b, sr_b)

        plsc.subcore_barrier()
        pltpu.sync_copy(acc_sh.at[pl.ds(s * rows, rows)],
                        out_hbm.at[c].at[pl.ds(s * rows, rows)])

        @pl.when(s == 0)
        def _():
            pltpu.sync_copy(acc_sh.at[pl.ds(tail_off, tail)],
                            out_hbm.at[c].at[pl.ds(tail_off, tail)])

    return sc_gather2, sc_scatter_add


def kernel(node_features, edge_features, edge_index, We1, be1, We2, be2,
           Wn1, bn1, Wn2, bn2):
    n = node_features
    e = edge_features
    src = edge_index[0].astype(jnp.int32)
    dst = edge_index[1].astype(jnp.int32)
    zero = jnp.zeros((N_NODES, 2 * EDGE_DIM), jnp.float32)
    L = We1.shape[0]
    for l in range(L):
        w1e = We1[l, :EDGE_DIM]
        w1s = We1[l, EDGE_DIM:EDGE_DIM + NODE_DIM]
        w1d = We1[l, EDGE_DIM + NODE_DIM:]
        b1 = be1[l].reshape(1, HIDDEN)
        w2 = We2[l]
        b2 = be2[l].reshape(1, EDGE_DIM)
        w1a = jnp.pad(Wn1[l, NODE_DIM:], ((0, EDGE_DIM), (0, 0)))
        sc_gather2, sc_scatter_add = _sc_kernels()
        ps, pd = _proj(n, w1s, w1d, b1)
        gs, gd = sc_gather2(ps, pd, src, dst)
        e = _edge_mlp(e, gs, gd, w1e, w2, b2)
        agg = sc_scatter_add(e, dst, zero)
        n = _node_mlp(n, agg[0], agg[1],
                      Wn1[l, :NODE_DIM], w1a,
                      bn1[l].reshape(1, HIDDEN), Wn2[l],
                      bn2[l].reshape(1, NODE_DIM))
    return n


# speedup vs baseline: 4.3286x; 1.0138x over previous
import functools

import jax
import jax.numpy as jnp
from jax import lax
from jax.experimental import pallas as pl
from jax.experimental.pallas import tpu as pltpu
from jax.experimental.pallas import tpu_sc as plsc

N_NODES = 10000
N_EDGES = 320000
NODE_DIM = 128
EDGE_DIM = 64
HIDDEN = 128

NC = 2
NS = 16
NW = NC * NS

GWIN = 128
HALVES = 2
E_HALF = N_EDGES // HALVES


def _proj_body(n_ref, ws_ref, wd_ref, b1_ref, ps_ref, pd_ref):
    n = n_ref[...]
    ps_ref[...] = jnp.dot(n, ws_ref[...], preferred_element_type=jnp.float32) + b1_ref[...]
    pd_ref[...] = jnp.dot(n, wd_ref[...], preferred_element_type=jnp.float32)


def _proj(n, ws, wd, b1):
    R = 2000
    return pl.pallas_call(
        _proj_body,
        grid=(N_NODES // R,),
        in_specs=[
            pl.BlockSpec((R, NODE_DIM), lambda i: (i, 0)),
            pl.BlockSpec((NODE_DIM, HIDDEN), lambda i: (0, 0)),
            pl.BlockSpec((NODE_DIM, HIDDEN), lambda i: (0, 0)),
            pl.BlockSpec((1, HIDDEN), lambda i: (0, 0)),
        ],
        out_specs=[
            pl.BlockSpec((R, HIDDEN), lambda i: (i, 0)),
            pl.BlockSpec((R, HIDDEN), lambda i: (i, 0)),
        ],
        out_shape=[
            jax.ShapeDtypeStruct((N_NODES, HIDDEN), jnp.float32),
            jax.ShapeDtypeStruct((N_NODES, HIDDEN), jnp.float32),
        ],
    )(n, ws, wd, b1)


def _edge_body(e_ref, gs_ref, gd_ref, w1_ref, w2_ref, b2_ref, o_ref):
    e = e_ref[...][:, :EDGE_DIM]
    h = jnp.dot(e, w1_ref[...], preferred_element_type=jnp.float32)
    h = jnp.maximum(h + gs_ref[...] + gd_ref[...], 0.0)
    o = e + jnp.dot(h, w2_ref[...], preferred_element_type=jnp.float32) + b2_ref[...]
    o_ref[...] = jnp.concatenate([o, jnp.zeros_like(o)], axis=1)


def _edge_mlp(e, gs, gd, w1e, w2, b2, e_row_base):
    R = 3200
    ew = e.shape[1]
    eb = e_row_base // R
    return pl.pallas_call(
        _edge_body,
        grid=(E_HALF // R,),
        in_specs=[
            pl.BlockSpec((R, ew), lambda i: (i + eb, 0)),
            pl.BlockSpec((R, HIDDEN), lambda i: (i, 0)),
            pl.BlockSpec((R, HIDDEN), lambda i: (i, 0)),
            pl.BlockSpec((EDGE_DIM, HIDDEN), lambda i: (0, 0)),
            pl.BlockSpec((HIDDEN, EDGE_DIM), lambda i: (0, 0)),
            pl.BlockSpec((1, EDGE_DIM), lambda i: (0, 0)),
        ],
        out_specs=pl.BlockSpec((R, 2 * EDGE_DIM), lambda i: (i, 0)),
        out_shape=jax.ShapeDtypeStruct((E_HALF, 2 * EDGE_DIM), jnp.float32),
    )(e, gs, gd, w1e, w2, b2)


def _node_body(n_ref, a0_ref, a1_ref, a2_ref, a3_ref,
               w1n_ref, w1a_ref, b1_ref, w2_ref, b2_ref, o_ref):
    n = n_ref[...]
    agg = a0_ref[...] + a1_ref[...] + a2_ref[...] + a3_ref[...]
    h = (jnp.dot(n, w1n_ref[...], preferred_element_type=jnp.float32)
         + jnp.dot(agg, w1a_ref[...], preferred_element_type=jnp.float32)
         + b1_ref[...])
    h = jnp.maximum(h, 0.0)
    o_ref[...] = n + jnp.dot(h, w2_ref[...], preferred_element_type=jnp.float32) + b2_ref[...]


def _node_mlp(n, aggs, w1n, w1a, b1, w2, b2):
    R = 2000
    return pl.pallas_call(
        _node_body,
        grid=(N_NODES // R,),
        in_specs=[
            pl.BlockSpec((R, NODE_DIM), lambda i: (i, 0)),
            pl.BlockSpec((R, 2 * EDGE_DIM), lambda i: (i, 0)),
            pl.BlockSpec((R, 2 * EDGE_DIM), lambda i: (i, 0)),
            pl.BlockSpec((R, 2 * EDGE_DIM), lambda i: (i, 0)),
            pl.BlockSpec((R, 2 * EDGE_DIM), lambda i: (i, 0)),
            pl.BlockSpec((NODE_DIM, HIDDEN), lambda i: (0, 0)),
            pl.BlockSpec((2 * EDGE_DIM, HIDDEN), lambda i: (0, 0)),
            pl.BlockSpec((1, HIDDEN), lambda i: (0, 0)),
            pl.BlockSpec((HIDDEN, NODE_DIM), lambda i: (0, 0)),
            pl.BlockSpec((1, NODE_DIM), lambda i: (0, 0)),
        ],
        out_specs=pl.BlockSpec((R, NODE_DIM), lambda i: (i, 0)),
        out_shape=jax.ShapeDtypeStruct((N_NODES, NODE_DIM), jnp.float32),
    )(n, aggs[0], aggs[1], aggs[2], aggs[3], w1n, w1a, b1, w2, b2)


@functools.cache
def _sc_kernels(base_row):
    mesh = plsc.VectorSubcoreMesh(core_axis_name="c", subcore_axis_name="s")
    grid = E_HALF // GWIN

    gather_scratch = []
    for _ in range(2):
        gather_scratch += [
            pltpu.VMEM((GWIN,), jnp.int32),
            pltpu.VMEM((GWIN,), jnp.int32),
            pltpu.VMEM((GWIN, HIDDEN), jnp.float32),
            pltpu.VMEM((GWIN, HIDDEN), jnp.float32),
        ] + [pltpu.SemaphoreType.DMA] * 6

    @functools.partial(
        pl.kernel,
        out_type=(
            jax.ShapeDtypeStruct((E_HALF, HIDDEN), jnp.float32),
            jax.ShapeDtypeStruct((E_HALF, HIDDEN), jnp.float32),
        ),
        mesh=mesh,
        scratch_types=gather_scratch,
    )
    def sc_gather2(ps_hbm, pd_hbm, src_hbm, dst_hbm, gs_hbm, gd_hbm, *scr):
        c = lax.axis_index("c")
        s = lax.axis_index("s")
        wid = s * NC + c
        bufs = (scr[:10], scr[10:])

        def issue_idx(j, buf):
            si_v, di_v = buf[0], buf[1]
            ssi, sdi = buf[4], buf[5]
            k = (j * NW + wid) * GWIN
            pltpu.async_copy(src_hbm.at[pl.ds(base_row + k, GWIN)], si_v, ssi)
            pltpu.async_copy(dst_hbm.at[pl.ds(base_row + k, GWIN)], di_v, sdi)

        issue_idx(0, bufs[0])
        issue_idx(1, bufs[1])

        @pl.loop(0, grid // (2 * NW) + 1)
        def _(i):
            for b in range(2):
                j = 2 * i + b
                k = (j * NW + wid) * GWIN
                si_v, di_v, gs_v, gd_v, ssi, sdi, sgs, sgd, sws, swd = bufs[b]

                @pl.when(j * NW + wid < grid)
                def _():
                    pltpu.make_async_copy(
                        src_hbm.at[pl.ds(base_row + k, GWIN)], si_v, ssi).wait()
                    pltpu.make_async_copy(
                        dst_hbm.at[pl.ds(base_row + k, GWIN)], di_v, sdi).wait()

                    @pl.when(i >= 1)
                    def _():
                        kprev = k - 2 * GWIN * NW
                        pltpu.make_async_copy(
                            gs_v, gs_hbm.at[pl.ds(kprev, GWIN)], sws).wait()
                        pltpu.make_async_copy(
                            gd_v, gd_hbm.at[pl.ds(kprev, GWIN)], swd).wait()

                    pltpu.async_copy(ps_hbm.at[si_v], gs_v, sgs)
                    pltpu.async_copy(pd_hbm.at[di_v], gd_v, sgd)
                    pltpu.make_async_copy(ps_hbm.at[si_v], gs_v, sgs).wait()
                    pltpu.make_async_copy(pd_hbm.at[di_v], gd_v, sgd).wait()
                    pltpu.async_copy(gs_v, gs_hbm.at[pl.ds(k, GWIN)], sws)
                    pltpu.async_copy(gd_v, gd_hbm.at[pl.ds(k, GWIN)], swd)

                    @pl.when((j + 2) * NW + wid < grid)
                    def _():
                        issue_idx(j + 2, bufs[b])

        for b in range(2):
            si_v, di_v, gs_v, gd_v, ssi, sdi, sgs, sgd, sws, swd = bufs[b]
            pltpu.make_async_copy(gs_v, gs_hbm.at[pl.ds(0, GWIN)], sws).wait()
            pltpu.make_async_copy(gd_v, gd_hbm.at[pl.ds(0, GWIN)], swd).wait()

    @functools.partial(
        pl.kernel,
        out_type=jax.ShapeDtypeStruct((NC, N_NODES, 2 * EDGE_DIM), jnp.float32),
        mesh=mesh,
        scratch_types=[
            pltpu.VMEM_SHARED((N_NODES, 2 * EDGE_DIM), jnp.float32),
            pltpu.VMEM((GWIN,), jnp.int32),
            pltpu.VMEM((GWIN, 2 * EDGE_DIM), jnp.float32),
            pltpu.VMEM((GWIN,), jnp.int32),
            pltpu.VMEM((GWIN, 2 * EDGE_DIM), jnp.float32),
            pltpu.SemaphoreType.DMA,
            pltpu.SemaphoreType.DMA,
            pltpu.SemaphoreType.DMA,
            pltpu.SemaphoreType.DMA,
        ],
    )
    def sc_scatter_add(e_hbm, dst_hbm, zero_hbm, out_hbm, acc_sh,
                       idx0_v, rows0_v, idx1_v, rows1_v,
                       si0, sr0, si1, sr1):
        c = lax.axis_index("c")
        s = lax.axis_index("s")
        wid = s * NC + c
        rows = 624
        tail_off = NS * rows
        tail = N_NODES - tail_off

        pltpu.sync_copy(zero_hbm.at[pl.ds(s * rows, rows)],
                        acc_sh.at[pl.ds(s * rows, rows)])

        @pl.when(s == 0)
        def _():
            pltpu.sync_copy(zero_hbm.at[pl.ds(tail_off, tail)],
                            acc_sh.at[pl.ds(tail_off, tail)])

        plsc.subcore_barrier()

        bufs = ((idx0_v, rows0_v, si0, sr0), (idx1_v, rows1_v, si1, sr1))

        def issue(j, idx_b, rows_b, si_b, sr_b):
            k = (j * NW + wid) * GWIN
            pltpu.async_copy(dst_hbm.at[pl.ds(base_row + k, GWIN)], idx_b, si_b)
            pltpu.async_copy(e_hbm.at[pl.ds(k, GWIN)], rows_b, sr_b)

        issue(0, *bufs[0])
        issue(1, *bufs[1])

        @pl.loop(0, grid // (2 * NW) + 1)
        def _(i):
            for b in range(2):
                j = 2 * i + b
                k = (j * NW + wid) * GWIN
                idx_b, rows_b, si_b, sr_b = bufs[b]

                @pl.when(j * NW + wid < grid)
                def _():
                    pltpu.make_async_copy(
                        dst_hbm.at[pl.ds(base_row + k, GWIN)], idx_b, si_b).wait()
                    pltpu.make_async_copy(
                        e_hbm.at[pl.ds(k, GWIN)], rows_b, sr_b).wait()
                    pltpu.sync_copy(rows_b, acc_sh.at[idx_b], add=True)

                    @pl.when((j + 2) * NW + wid < grid)
                    def _():
                        issue(j + 2, idx_b, rows_b, si_b, sr_b)

        plsc.subcore_barrier()
        pltpu.sync_copy(acc_sh.at[pl.ds(s * rows, rows)],
                        out_hbm.at[c].at[pl.ds(s * rows, rows)])

        @pl.when(s == 0)
        def _():
            pltpu.sync_copy(acc_sh.at[pl.ds(tail_off, tail)],
                            out_hbm.at[c].at[pl.ds(tail_off, tail)])

    return sc_gather2, sc_scatter_add


def kernel(node_features, edge_features, edge_index, We1, be1, We2, be2,
           Wn1, bn1, Wn2, bn2):
    n = node_features
    src = edge_index[0].astype(jnp.int32)
    dst = edge_index[1].astype(jnp.int32)
    zero = jnp.zeros((N_NODES, 2 * EDGE_DIM), jnp.float32)
    e_halves = [edge_features] * HALVES
    e_bases = [h * E_HALF for h in range(HALVES)]
    L = We1.shape[0]
    for l in range(L):
        w1e = We1[l, :EDGE_DIM]
        w1s = We1[l, EDGE_DIM:EDGE_DIM + NODE_DIM]
        w1d = We1[l, EDGE_DIM + NODE_DIM:]
        b1 = be1[l].reshape(1, HIDDEN)
        w2 = We2[l]
        b2 = be2[l].reshape(1, EDGE_DIM)
        w1a = jnp.pad(Wn1[l, NODE_DIM:], ((0, EDGE_DIM), (0, 0)))

        ps, pd = _proj(n, w1s, w1d, b1)
        new_e, aggs = [], []
        for h in range(HALVES):
            sc_gather2, sc_scatter_add = _sc_kernels(h * E_HALF)
            gs, gd = sc_gather2(ps, pd, src, dst)
            eh = _edge_mlp(e_halves[h], gs, gd, w1e, w2, b2, e_bases[h])
            agg = sc_scatter_add(eh, dst, zero)
            new_e.append(eh)
            aggs += [agg[0], agg[1]]
        e_halves = new_e
        e_bases = [0] * HALVES
        n = _node_mlp(n, aggs, Wn1[l, :NODE_DIM], w1a,
                      bn1[l].reshape(1, HIDDEN), Wn2[l],
                      bn2[l].reshape(1, NODE_DIM))
    return n
